# baseline (speedup 1.0000x reference)
import functools

import jax
import jax.numpy as jnp
from jax import lax
from jax.experimental import pallas as pl
from jax.experimental.pallas import tpu as pltpu
from jax.experimental.pallas import tpu_sc as plsc

_B = 16384
_D = 64
_L = 16
_HALF = 8192
_NC = 2
_NS = 16
_NW = _NC * _NS
_RPW = _B // _NW
_CHUNK = 128
_NCHUNK = _RPW // _CHUNK


def _sc_body(h_hbm, t_hbm, r_hbm, ent_hbm, rel_hbm, out_hbm,
             idx_h, idx_t, idx_r, rows_h, rows_t, rows_r, p_v, sem):
    wid = lax.axis_index("s") * _NC + lax.axis_index("c")

    pltpu.sync_copy(h_hbm.at[wid], idx_h)
    pltpu.sync_copy(t_hbm.at[wid], idx_t)
    pltpu.sync_copy(r_hbm.at[wid], idx_r)

    copies = []
    for c in range(_NCHUNK):
        sl = pl.ds(c * _CHUNK, _CHUNK)
        copies.append(pltpu.async_copy(ent_hbm.at[idx_h.at[c]], rows_h.at[sl], sem))
        copies.append(pltpu.async_copy(ent_hbm.at[idx_t.at[c]], rows_t.at[sl], sem))
        copies.append(pltpu.async_copy(rel_hbm.at[idx_r.at[c]], rows_r.at[sl], sem))
    for cp in copies:
        cp.wait()

    def row(r, _):
        s = None
        for k in range(_D // _L):
            ksl = pl.ds(k * _L, _L)
            d = rows_h[r, ksl] + rows_r[r, ksl] - rows_t[r, ksl]
            sq = d * d
            s = sq if s is None else s + sq
        p_v[pl.ds(r * _L, _L)] = s
        return 0

    lax.fori_loop(0, _RPW, row, 0)
    pltpu.sync_copy(p_v, out_hbm.at[pl.ds(wid * _RPW * _L, _RPW * _L)])


@functools.partial(
    pl.kernel,
    out_type=jax.ShapeDtypeStruct((_B * _L,), jnp.float32),
    mesh=plsc.VectorSubcoreMesh(core_axis_name="c", subcore_axis_name="s"),
    compiler_params=pltpu.CompilerParams(use_tc_tiling_on_sc=False),
    scratch_types=[
        pltpu.VMEM((_NCHUNK, _CHUNK), jnp.int32),
        pltpu.VMEM((_NCHUNK, _CHUNK), jnp.int32),
        pltpu.VMEM((_NCHUNK, _CHUNK), jnp.int32),
        pltpu.VMEM((_RPW, _D), jnp.float32),
        pltpu.VMEM((_RPW, _D), jnp.float32),
        pltpu.VMEM((_RPW, _D), jnp.float32),
        pltpu.VMEM((_RPW * _L,), jnp.float32),
        pltpu.SemaphoreType.DMA,
    ],
)
def _transe_partials(h_hbm, t_hbm, r_hbm, ent_hbm, rel_hbm, out_hbm,
                     idx_h, idx_t, idx_r, rows_h, rows_t, rows_r, p_v, sem):
    _sc_body(h_hbm, t_hbm, r_hbm, ent_hbm, rel_hbm, out_hbm,
             idx_h, idx_t, idx_r, rows_h, rows_t, rows_r, p_v, sem)


def _fold_body(p_ref, o_ref):
    o_ref[...] = jnp.sqrt(jnp.sum(p_ref[...], axis=-1))


_fold_sqrt = pl.pallas_call(
    _fold_body,
    out_shape=jax.ShapeDtypeStruct((_B,), jnp.float32),
)


def kernel(h, r, t, batch_size, ent_emb, rel_emb):
    del batch_size
    h3 = h.astype(jnp.int32).reshape(_NW, _NCHUNK, _CHUNK)
    t3 = t.astype(jnp.int32).reshape(_NW, _NCHUNK, _CHUNK)
    r3 = r.astype(jnp.int32).reshape(_NW, _NCHUNK, _CHUNK)
    partials = _transe_partials(h3, t3, r3, ent_emb, rel_emb)
    score = _fold_sqrt(partials.reshape(_B, _L))
    return score[:_HALF], score[_HALF:]
